# baseline (speedup 1.0000x reference)
import functools

import jax
import jax.numpy as jnp
from jax import lax
from jax.experimental import pallas as pl
from jax.experimental.pallas import tpu as pltpu
from jax.experimental.pallas import tpu_sc as plsc

N = 10000
E = 320000
D = 128
REL = 500
RPAD = 512
NPAD = 10112
ACCW = 144
NC = 2
NS = 16
NW = NC * NS
EPT = E // NW
K = 80
NCH = EPT // K
RPT = NPAD // NS


def _proj_body(x_ref, rel_ref, w1_ref, b1_ref, w2_ref,
               ps_ref, pd_ref, pr_ref, u_ref, v_ref, rv_ref):
    dn = (((1,), (1,)), ((), ()))
    x = x_ref[...]
    w1 = w1_ref[...]
    w2 = w2_ref[...]
    ps = lax.dot_general(x, w1[:, :D], dn, preferred_element_type=jnp.float32)
    pd = lax.dot_general(x, w1[:, D:2 * D], dn,
                         preferred_element_type=jnp.float32) + b1_ref[...]
    pr = lax.dot_general(rel_ref[...], w1[:, 2 * D:], dn,
                         preferred_element_type=jnp.float32)
    ps_ref[...] = ps
    pd_ref[...] = pd
    pr_ref[...] = pr
    u_ref[...] = lax.dot_general(ps, w2, dn, preferred_element_type=jnp.float32)
    v_ref[...] = lax.dot_general(pd, w2, dn, preferred_element_type=jnp.float32)
    rv_ref[...] = lax.dot_general(pr, w2, dn, preferred_element_type=jnp.float32)


def _projections(x, rel, w1_w, w1_b, w2_w):
    return pl.pallas_call(
        _proj_body,
        out_shape=(
            jax.ShapeDtypeStruct((N, D), jnp.float32),
            jax.ShapeDtypeStruct((N, D), jnp.float32),
            jax.ShapeDtypeStruct((REL, D), jnp.float32),
            jax.ShapeDtypeStruct((N, 1), jnp.float32),
            jax.ShapeDtypeStruct((N, 1), jnp.float32),
            jax.ShapeDtypeStruct((REL, 1), jnp.float32),
        ),
    )(x, rel, w1_w, w1_b.reshape(1, D), w2_w)


def _edge_body(ps_hbm, pr_hbm, u_hbm, v_hbm, r_hbm, src_hbm, dst_hbm, typ_hbm,
               z128_hbm, z16_hbm, o128_hbm, o16_hbm,
               uvals, vvals, rvals, src2, dst_v, typ_v, exp_v, psr, prr, dexp,
               acc_sh, accd_sh, pr_sh, u_sh, v_sh, r_sh, sem_ps, sem2, sem3):
    sid = lax.axis_index("s")
    cid = lax.axis_index("c")
    wid = sid * NC + cid

    @pl.when(sid == 0)
    def _():
        pltpu.sync_copy(pr_hbm, pr_sh)
        pltpu.sync_copy(u_hbm, u_sh)
        pltpu.sync_copy(v_hbm, v_sh)
        pltpu.sync_copy(r_hbm, r_sh)

    rsl = pl.ds(sid * RPT, RPT)
    pltpu.sync_copy(z128_hbm.at[rsl], acc_sh.at[rsl])
    pltpu.sync_copy(z16_hbm.at[rsl], accd_sh.at[rsl])
    plsc.subcore_barrier()

    lane0 = jnp.where(lax.iota(jnp.int32, 16) == 0,
                      jnp.float32(1.0), jnp.float32(0.0))
    ebase = wid * EPT

    pltpu.sync_copy(src_hbm.at[pl.ds(ebase, K)], src2.at[pl.ds(0, K)])
    pltpu.async_copy(ps_hbm.at[src2.at[pl.ds(0, K)]], psr.at[pl.ds(0, K)],
                     sem_ps)

    def chunk(ci, carry):
        base = ebase + ci * K
        cur = (ci % 2) * K
        nxt = ((ci + 1) % 2) * K

        @pl.when(ci + 1 < NCH)
        def _():
            pltpu.sync_copy(src_hbm.at[pl.ds(base + K, K)],
                            src2.at[pl.ds(nxt, K)])
            pltpu.async_copy(ps_hbm.at[src2.at[pl.ds(nxt, K)]],
                             psr.at[pl.ds(nxt, K)], sem_ps)

        pltpu.sync_copy(dst_hbm.at[pl.ds(base, K)], dst_v)
        pltpu.sync_copy(typ_hbm.at[pl.ds(base, K)], typ_v)
        pltpu.async_copy(pr_hbm.at[typ_v], prr, sem2)
        pltpu.async_copy(u_sh.at[src2.at[pl.ds(cur, K)]], uvals, sem3)
        pltpu.async_copy(v_sh.at[dst_v], vvals, sem3)
        pltpu.async_copy(r_sh.at[typ_v], rvals, sem3)
        pltpu.make_async_copy(u_sh.at[src2.at[pl.ds(cur, K)]], uvals,
                              sem3).wait()
        pltpu.make_async_copy(v_sh.at[dst_v], vvals, sem3).wait()
        pltpu.make_async_copy(r_sh.at[typ_v], rvals, sem3).wait()

        def score(g, c2):
            gs = pl.ds(g * 16, 16)
            b = uvals[gs] + vvals[gs] + rvals[gs]
            b = jnp.where(b >= 0, b, b * jnp.float32(0.01))
            exp_v[gs] = jnp.exp(b)
            return c2

        lax.fori_loop(0, K // 16, score, 0, unroll=True)

        pltpu.make_async_copy(ps_hbm.at[src2.at[pl.ds(cur, K)]],
                              psr.at[pl.ds(cur, K)], sem_ps).wait()
        pltpu.make_async_copy(pr_hbm.at[typ_v], prr, sem2).wait()

        def emit(g, c2):
            e16 = exp_v[pl.ds(g * 16, 16)]
            for k2 in range(16):
                k = g * 16 + k2
                s = jnp.full((16,), e16[k2], jnp.float32)
                for j in range(8):
                    sl = pl.ds(j * 16, 16)
                    psr[cur + k, sl] = s * (psr[cur + k, sl] + prr[k, sl])
                dexp[k, pl.ds(0, 16)] = s * lane0
            return c2

        lax.fori_loop(0, K // 16, emit, 0)
        pltpu.sync_copy(psr.at[pl.ds(cur, K)], acc_sh.at[dst_v], add=True)
        pltpu.sync_copy(dexp, accd_sh.at[dst_v], add=True)
        return carry

    lax.fori_loop(0, NCH, chunk, 0)
    plsc.subcore_barrier()
    pltpu.sync_copy(acc_sh.at[rsl], o128_hbm.at[cid].at[rsl])
    pltpu.sync_copy(accd_sh.at[rsl], o16_hbm.at[cid].at[rsl])


def _edge_pass(ps, pr, u, v, r, src, dst, typ, z128, z16):
    mesh = plsc.VectorSubcoreMesh(core_axis_name="c", subcore_axis_name="s")
    f = functools.partial(
        pl.kernel,
        mesh=mesh,
        compiler_params=pltpu.CompilerParams(
            needs_layout_passes=False, use_tc_tiling_on_sc=False),
        out_type=(
            jax.ShapeDtypeStruct((NC, NPAD, D), jnp.float32),
            jax.ShapeDtypeStruct((NC, NPAD, 16), jnp.float32),
        ),
        scratch_types=[
            pltpu.VMEM((K,), jnp.float32),
            pltpu.VMEM((K,), jnp.float32),
            pltpu.VMEM((K,), jnp.float32),
            pltpu.VMEM((2 * K,), jnp.int32),
            pltpu.VMEM((K,), jnp.int32),
            pltpu.VMEM((K,), jnp.int32),
            pltpu.VMEM((K,), jnp.float32),
            pltpu.VMEM((2 * K, D), jnp.float32),
            pltpu.VMEM((K, D), jnp.float32),
            pltpu.VMEM((K, 16), jnp.float32),
            pltpu.VMEM_SHARED((NPAD, D), jnp.float32),
            pltpu.VMEM_SHARED((NPAD, 16), jnp.float32),
            pltpu.VMEM_SHARED((REL, D), jnp.float32),
            pltpu.VMEM_SHARED((N,), jnp.float32),
            pltpu.VMEM_SHARED((N,), jnp.float32),
            pltpu.VMEM_SHARED((RPAD,), jnp.float32),
            pltpu.SemaphoreType.DMA,
            pltpu.SemaphoreType.DMA,
            pltpu.SemaphoreType.DMA,
        ],
    )(_edge_body)
    return f(ps, pr, u, v, r, src, dst, typ, z128, z16)


def _fin_body(a128_ref, a16_ref, pd_ref, o_ref):
    s = a128_ref[0, :N] + a128_ref[1, :N]
    d = a16_ref[0, :N, :1] + a16_ref[1, :N, :1]
    safe = jnp.where(d > 0, d, jnp.float32(1.0))
    y = s / safe + pd_ref[...]
    y = jnp.where(y >= 0, y, y * jnp.float32(0.01))
    o_ref[...] = jnp.where(d > 0, y, jnp.float32(0.0))


def _finish(a128, a16, pd):
    return pl.pallas_call(
        _fin_body,
        out_shape=jax.ShapeDtypeStruct((N, D), jnp.float32),
    )(a128, a16, pd)


def kernel(x, relation_embedding, w1_w, w1_b, w2_w, edge_index, edge_type):
    ps, pd, pr, u, v, rv = _projections(x, relation_embedding, w1_w, w1_b, w2_w)
    r_pad = jnp.pad(rv[:, 0], (0, RPAD - REL))
    z128 = jnp.zeros((NPAD, D), jnp.float32)
    z16 = jnp.zeros((NPAD, 16), jnp.float32)
    a128, a16 = _edge_pass(ps, pr, u[:, 0], v[:, 0], r_pad,
                           edge_index[0], edge_index[1], edge_type, z128, z16)
    return _finish(a128, a16, pd)


# speedup vs baseline: 1.0703x; 1.0703x over previous
import functools

import jax
import jax.numpy as jnp
from jax import lax
from jax.experimental import pallas as pl
from jax.experimental.pallas import tpu as pltpu
from jax.experimental.pallas import tpu_sc as plsc

N = 10000
E = 320000
D = 128
REL = 500
RPAD = 512
NPAD = 10112
ACCW = 144
NC = 2
NS = 16
NW = NC * NS
EPT = E // NW
K = 80
NCH = EPT // K
RPT = NPAD // NS


def _proj_body(x_ref, rel_ref, w1_ref, b1_ref, w2_ref,
               ps_ref, pd_ref, pr_ref, u_ref, v_ref, rv_ref):
    dn = (((1,), (1,)), ((), ()))
    x = x_ref[...]
    w1 = w1_ref[...]
    w2 = w2_ref[...]
    ps = lax.dot_general(x, w1[:, :D], dn, preferred_element_type=jnp.float32)
    pd = lax.dot_general(x, w1[:, D:2 * D], dn,
                         preferred_element_type=jnp.float32) + b1_ref[...]
    pr = lax.dot_general(rel_ref[...], w1[:, 2 * D:], dn,
                         preferred_element_type=jnp.float32)
    ps_ref[...] = ps
    pd_ref[...] = pd
    pr_ref[...] = pr
    u_ref[...] = lax.dot_general(ps, w2, dn, preferred_element_type=jnp.float32)
    v_ref[...] = lax.dot_general(pd, w2, dn, preferred_element_type=jnp.float32)
    rv_ref[...] = lax.dot_general(pr, w2, dn, preferred_element_type=jnp.float32)


def _projections(x, rel, w1_w, w1_b, w2_w):
    return pl.pallas_call(
        _proj_body,
        out_shape=(
            jax.ShapeDtypeStruct((N, D), jnp.float32),
            jax.ShapeDtypeStruct((N, D), jnp.float32),
            jax.ShapeDtypeStruct((REL, D), jnp.float32),
            jax.ShapeDtypeStruct((N, 1), jnp.float32),
            jax.ShapeDtypeStruct((N, 1), jnp.float32),
            jax.ShapeDtypeStruct((REL, 1), jnp.float32),
        ),
    )(x, rel, w1_w, w1_b.reshape(1, D), w2_w)


def _edge_body(ps_hbm, pr_hbm, u_hbm, v_hbm, r_hbm, src_hbm, dst_hbm, typ_hbm,
               z128_hbm, z16_hbm, o128_hbm, o16_hbm,
               uvals, vvals, rvals, src2, dst_v, typ_v, exp_v, psr, prr, dexp,
               acc_sh, accd_sh, pr_sh, u_sh, v_sh, r_sh, sem_ps, sem2, sem3,
               sem_sc, sem_sd):
    sid = lax.axis_index("s")
    cid = lax.axis_index("c")
    wid = sid * NC + cid

    @pl.when(sid == 0)
    def _():
        pltpu.sync_copy(pr_hbm, pr_sh)
        pltpu.sync_copy(u_hbm, u_sh)
        pltpu.sync_copy(v_hbm, v_sh)
        pltpu.sync_copy(r_hbm, r_sh)

    rsl = pl.ds(sid * RPT, RPT)
    pltpu.sync_copy(z128_hbm.at[rsl], acc_sh.at[rsl])
    pltpu.sync_copy(z16_hbm.at[rsl], accd_sh.at[rsl])
    plsc.subcore_barrier()

    lane0 = jnp.where(lax.iota(jnp.int32, 16) == 0,
                      jnp.float32(1.0), jnp.float32(0.0))
    ebase = wid * EPT

    pltpu.sync_copy(src_hbm.at[pl.ds(ebase, K)], src2.at[pl.ds(0, K)])
    pltpu.async_copy(ps_hbm.at[src2.at[pl.ds(0, K)]], psr.at[pl.ds(0, K)],
                     sem_ps)

    def chunk(ci, carry):
        base = ebase + ci * K
        cur = (ci % 2) * K
        nxt = ((ci + 1) % 2) * K

        @pl.when(ci > 0)
        def _():
            pltpu.make_async_copy(psr.at[pl.ds(nxt, K)], acc_sh.at[dst_v],
                                  sem_sc).wait()
            pltpu.make_async_copy(dexp.at[pl.ds(nxt, K)], accd_sh.at[dst_v],
                                  sem_sd).wait()

        @pl.when(ci + 1 < NCH)
        def _():
            pltpu.sync_copy(src_hbm.at[pl.ds(base + K, K)],
                            src2.at[pl.ds(nxt, K)])
            pltpu.async_copy(ps_hbm.at[src2.at[pl.ds(nxt, K)]],
                             psr.at[pl.ds(nxt, K)], sem_ps)

        pltpu.sync_copy(dst_hbm.at[pl.ds(base, K)], dst_v)
        pltpu.sync_copy(typ_hbm.at[pl.ds(base, K)], typ_v)
        pltpu.async_copy(pr_sh.at[typ_v], prr, sem2)
        pltpu.async_copy(u_sh.at[src2.at[pl.ds(cur, K)]], uvals, sem3)
        pltpu.async_copy(v_sh.at[dst_v], vvals, sem3)
        pltpu.async_copy(r_sh.at[typ_v], rvals, sem3)
        pltpu.make_async_copy(u_sh.at[src2.at[pl.ds(cur, K)]], uvals,
                              sem3).wait()
        pltpu.make_async_copy(v_sh.at[dst_v], vvals, sem3).wait()
        pltpu.make_async_copy(r_sh.at[typ_v], rvals, sem3).wait()

        def score(g, c2):
            gs = pl.ds(g * 16, 16)
            b = uvals[gs] + vvals[gs] + rvals[gs]
            b = jnp.where(b >= 0, b, b * jnp.float32(0.01))
            exp_v[gs] = jnp.exp(b)
            return c2

        lax.fori_loop(0, K // 16, score, 0, unroll=True)

        pltpu.make_async_copy(ps_hbm.at[src2.at[pl.ds(cur, K)]],
                              psr.at[pl.ds(cur, K)], sem_ps).wait()
        pltpu.make_async_copy(pr_sh.at[typ_v], prr, sem2).wait()

        def emit(g, c2):
            e16 = exp_v[pl.ds(g * 16, 16)]
            for k2 in range(16):
                k = g * 16 + k2
                s = jnp.full((16,), e16[k2], jnp.float32)
                for j in range(8):
                    sl = pl.ds(j * 16, 16)
                    psr[cur + k, sl] = s * (psr[cur + k, sl] + prr[k, sl])
                dexp[cur + k, pl.ds(0, 16)] = s * lane0
            return c2

        lax.fori_loop(0, K // 16, emit, 0)
        pltpu.async_copy(psr.at[pl.ds(cur, K)], acc_sh.at[dst_v], sem_sc,
                         add=True)
        pltpu.async_copy(dexp.at[pl.ds(cur, K)], accd_sh.at[dst_v], sem_sd,
                         add=True)
        return carry

    lax.fori_loop(0, NCH, chunk, 0)
    last = ((NCH - 1) % 2) * K
    pltpu.make_async_copy(psr.at[pl.ds(last, K)], acc_sh.at[dst_v],
                          sem_sc).wait()
    pltpu.make_async_copy(dexp.at[pl.ds(last, K)], accd_sh.at[dst_v],
                          sem_sd).wait()
    plsc.subcore_barrier()
    pltpu.sync_copy(acc_sh.at[rsl], o128_hbm.at[cid].at[rsl])
    pltpu.sync_copy(accd_sh.at[rsl], o16_hbm.at[cid].at[rsl])


def _edge_pass(ps, pr, u, v, r, src, dst, typ, z128, z16):
    mesh = plsc.VectorSubcoreMesh(core_axis_name="c", subcore_axis_name="s")
    f = functools.partial(
        pl.kernel,
        mesh=mesh,
        compiler_params=pltpu.CompilerParams(
            needs_layout_passes=False, use_tc_tiling_on_sc=False),
        out_type=(
            jax.ShapeDtypeStruct((NC, NPAD, D), jnp.float32),
            jax.ShapeDtypeStruct((NC, NPAD, 16), jnp.float32),
        ),
        scratch_types=[
            pltpu.VMEM((K,), jnp.float32),
            pltpu.VMEM((K,), jnp.float32),
            pltpu.VMEM((K,), jnp.float32),
            pltpu.VMEM((2 * K,), jnp.int32),
            pltpu.VMEM((K,), jnp.int32),
            pltpu.VMEM((K,), jnp.int32),
            pltpu.VMEM((K,), jnp.float32),
            pltpu.VMEM((2 * K, D), jnp.float32),
            pltpu.VMEM((K, D), jnp.float32),
            pltpu.VMEM((2 * K, 16), jnp.float32),
            pltpu.VMEM_SHARED((NPAD, D), jnp.float32),
            pltpu.VMEM_SHARED((NPAD, 16), jnp.float32),
            pltpu.VMEM_SHARED((REL, D), jnp.float32),
            pltpu.VMEM_SHARED((N,), jnp.float32),
            pltpu.VMEM_SHARED((N,), jnp.float32),
            pltpu.VMEM_SHARED((RPAD,), jnp.float32),
            pltpu.SemaphoreType.DMA,
            pltpu.SemaphoreType.DMA,
            pltpu.SemaphoreType.DMA,
            pltpu.SemaphoreType.DMA,
            pltpu.SemaphoreType.DMA,
        ],
    )(_edge_body)
    return f(ps, pr, u, v, r, src, dst, typ, z128, z16)


def _fin_body(a128_ref, a16_ref, pd_ref, o_ref):
    s = a128_ref[0, :N] + a128_ref[1, :N]
    d = a16_ref[0, :N, :1] + a16_ref[1, :N, :1]
    safe = jnp.where(d > 0, d, jnp.float32(1.0))
    y = s / safe + pd_ref[...]
    y = jnp.where(y >= 0, y, y * jnp.float32(0.01))
    o_ref[...] = jnp.where(d > 0, y, jnp.float32(0.0))


def _finish(a128, a16, pd):
    return pl.pallas_call(
        _fin_body,
        out_shape=jax.ShapeDtypeStruct((N, D), jnp.float32),
    )(a128, a16, pd)


def kernel(x, relation_embedding, w1_w, w1_b, w2_w, edge_index, edge_type):
    ps, pd, pr, u, v, rv = _projections(x, relation_embedding, w1_w, w1_b, w2_w)
    r_pad = jnp.pad(rv[:, 0], (0, RPAD - REL))
    z128 = jnp.zeros((NPAD, D), jnp.float32)
    z16 = jnp.zeros((NPAD, 16), jnp.float32)
    a128, a16 = _edge_pass(ps, pr, u[:, 0], v[:, 0], r_pad,
                           edge_index[0], edge_index[1], edge_type, z128, z16)
    return _finish(a128, a16, pd)


# speedup vs baseline: 1.0714x; 1.0010x over previous
import functools

import jax
import jax.numpy as jnp
from jax import lax
from jax.experimental import pallas as pl
from jax.experimental.pallas import tpu as pltpu
from jax.experimental.pallas import tpu_sc as plsc

N = 10000
E = 320000
D = 128
REL = 500
RPAD = 512
NPAD = 10112
ACCW = 144
NC = 2
NS = 16
NW = NC * NS
EPT = E // NW
K = 80
NCH = EPT // K
RPT = NPAD // NS


def _proj_body(x_ref, rel_ref, w1_ref, b1_ref, w2_ref,
               ps_ref, pd_ref, pr_ref, u_ref, v_ref, rv_ref):
    dn = (((1,), (1,)), ((), ()))
    x = x_ref[...]
    w1 = w1_ref[...]
    w2 = w2_ref[...]
    ps = lax.dot_general(x, w1[:, :D], dn, preferred_element_type=jnp.float32)
    pd = lax.dot_general(x, w1[:, D:2 * D], dn,
                         preferred_element_type=jnp.float32) + b1_ref[...]
    pr = lax.dot_general(rel_ref[...], w1[:, 2 * D:], dn,
                         preferred_element_type=jnp.float32)
    ps_ref[...] = ps
    pd_ref[...] = pd
    pr_ref[...] = pr
    u_ref[...] = lax.dot_general(ps, w2, dn, preferred_element_type=jnp.float32)
    v_ref[...] = lax.dot_general(pd, w2, dn, preferred_element_type=jnp.float32)
    rv_ref[...] = lax.dot_general(pr, w2, dn, preferred_element_type=jnp.float32)


def _projections(x, rel, w1_w, w1_b, w2_w):
    return pl.pallas_call(
        _proj_body,
        out_shape=(
            jax.ShapeDtypeStruct((N, D), jnp.float32),
            jax.ShapeDtypeStruct((N, D), jnp.float32),
            jax.ShapeDtypeStruct((REL, D), jnp.float32),
            jax.ShapeDtypeStruct((N, 1), jnp.float32),
            jax.ShapeDtypeStruct((N, 1), jnp.float32),
            jax.ShapeDtypeStruct((REL, 1), jnp.float32),
        ),
    )(x, rel, w1_w, w1_b.reshape(1, D), w2_w)


def _edge_body(ps_hbm, pr_hbm, u_hbm, v_hbm, r_hbm, src_hbm, dst_hbm, typ_hbm,
               z128_hbm, z16_hbm, o128_hbm, o16_hbm,
               uvals, vvals, rvals, src2, dst_v, typ_v, exp_v, psr, prr, dexp,
               acc_sh, accd_sh, pr_sh, u_sh, v_sh, r_sh, sem_ps, sem2, sem3,
               sem_sc, sem_sd):
    sid = lax.axis_index("s")
    cid = lax.axis_index("c")
    wid = sid * NC + cid

    @pl.when(sid == 0)
    def _():
        pltpu.sync_copy(pr_hbm, pr_sh)
        pltpu.sync_copy(u_hbm, u_sh)
        pltpu.sync_copy(v_hbm, v_sh)
        pltpu.sync_copy(r_hbm, r_sh)

    rsl = pl.ds(sid * RPT, RPT)
    pltpu.sync_copy(z128_hbm.at[rsl], acc_sh.at[rsl])
    pltpu.sync_copy(z16_hbm.at[rsl], accd_sh.at[rsl])
    plsc.subcore_barrier()

    lane0 = jnp.where(lax.iota(jnp.int32, 16) == 0,
                      jnp.float32(1.0), jnp.float32(0.0))
    ebase = wid * EPT

    pltpu.sync_copy(src_hbm.at[pl.ds(ebase, K)], src2.at[pl.ds(0, K)])
    pltpu.async_copy(ps_hbm.at[src2.at[pl.ds(0, K)]], psr.at[pl.ds(0, K)],
                     sem_ps)

    def chunk(ci, carry):
        base = ebase + ci * K
        cur = (ci % 2) * K
        nxt = ((ci + 1) % 2) * K

        @pl.when(ci > 0)
        def _():
            pltpu.make_async_copy(psr.at[pl.ds(nxt, K)], acc_sh.at[dst_v],
                                  sem_sc).wait()
            pltpu.make_async_copy(dexp.at[pl.ds(nxt, K)], accd_sh.at[dst_v],
                                  sem_sd).wait()

        @pl.when(ci + 1 < NCH)
        def _():
            pltpu.sync_copy(src_hbm.at[pl.ds(base + K, K)],
                            src2.at[pl.ds(nxt, K)])
            pltpu.async_copy(ps_hbm.at[src2.at[pl.ds(nxt, K)]],
                             psr.at[pl.ds(nxt, K)], sem_ps)

        pltpu.sync_copy(dst_hbm.at[pl.ds(base, K)], dst_v)
        pltpu.sync_copy(typ_hbm.at[pl.ds(base, K)], typ_v)
        pltpu.async_copy(pr_sh.at[typ_v], prr, sem2)
        pltpu.async_copy(u_sh.at[src2.at[pl.ds(cur, K)]], uvals, sem3)
        pltpu.async_copy(v_sh.at[dst_v], vvals, sem3)
        pltpu.async_copy(r_sh.at[typ_v], rvals, sem3)
        pltpu.make_async_copy(u_sh.at[src2.at[pl.ds(cur, K)]], uvals,
                              sem3).wait()
        pltpu.make_async_copy(v_sh.at[dst_v], vvals, sem3).wait()
        pltpu.make_async_copy(r_sh.at[typ_v], rvals, sem3).wait()

        def score(g, c2):
            gs = pl.ds(g * 16, 16)
            b = uvals[gs] + vvals[gs] + rvals[gs]
            b = jnp.where(b >= 0, b, b * jnp.float32(0.01))
            exp_v[gs] = jnp.exp(b)
            return c2

        lax.fori_loop(0, K // 16, score, 0, unroll=True)

        pltpu.make_async_copy(ps_hbm.at[src2.at[pl.ds(cur, K)]],
                              psr.at[pl.ds(cur, K)], sem_ps).wait()
        pltpu.make_async_copy(pr_sh.at[typ_v], prr, sem2).wait()

        @plsc.parallel_loop(0, K // 16, 1, unroll=2)
        def _(g):
            e16 = exp_v[pl.ds(g * 16, 16)]
            for k2 in range(16):
                k = g * 16 + k2
                s = jnp.full((16,), e16[k2], jnp.float32)
                for j in range(8):
                    sl = pl.ds(j * 16, 16)
                    psr[cur + k, sl] = s * (psr[cur + k, sl] + prr[k, sl])
                dexp[cur + k, pl.ds(0, 16)] = s * lane0
        pltpu.async_copy(psr.at[pl.ds(cur, K)], acc_sh.at[dst_v], sem_sc,
                         add=True)
        pltpu.async_copy(dexp.at[pl.ds(cur, K)], accd_sh.at[dst_v], sem_sd,
                         add=True)
        return carry

    lax.fori_loop(0, NCH, chunk, 0)
    last = ((NCH - 1) % 2) * K
    pltpu.make_async_copy(psr.at[pl.ds(last, K)], acc_sh.at[dst_v],
                          sem_sc).wait()
    pltpu.make_async_copy(dexp.at[pl.ds(last, K)], accd_sh.at[dst_v],
                          sem_sd).wait()
    plsc.subcore_barrier()
    pltpu.sync_copy(acc_sh.at[rsl], o128_hbm.at[cid].at[rsl])
    pltpu.sync_copy(accd_sh.at[rsl], o16_hbm.at[cid].at[rsl])


def _edge_pass(ps, pr, u, v, r, src, dst, typ, z128, z16):
    mesh = plsc.VectorSubcoreMesh(core_axis_name="c", subcore_axis_name="s")
    f = functools.partial(
        pl.kernel,
        mesh=mesh,
        compiler_params=pltpu.CompilerParams(
            needs_layout_passes=False, use_tc_tiling_on_sc=False),
        out_type=(
            jax.ShapeDtypeStruct((NC, NPAD, D), jnp.float32),
            jax.ShapeDtypeStruct((NC, NPAD, 16), jnp.float32),
        ),
        scratch_types=[
            pltpu.VMEM((K,), jnp.float32),
            pltpu.VMEM((K,), jnp.float32),
            pltpu.VMEM((K,), jnp.float32),
            pltpu.VMEM((2 * K,), jnp.int32),
            pltpu.VMEM((K,), jnp.int32),
            pltpu.VMEM((K,), jnp.int32),
            pltpu.VMEM((K,), jnp.float32),
            pltpu.VMEM((2 * K, D), jnp.float32),
            pltpu.VMEM((K, D), jnp.float32),
            pltpu.VMEM((2 * K, 16), jnp.float32),
            pltpu.VMEM_SHARED((NPAD, D), jnp.float32),
            pltpu.VMEM_SHARED((NPAD, 16), jnp.float32),
            pltpu.VMEM_SHARED((REL, D), jnp.float32),
            pltpu.VMEM_SHARED((N,), jnp.float32),
            pltpu.VMEM_SHARED((N,), jnp.float32),
            pltpu.VMEM_SHARED((RPAD,), jnp.float32),
            pltpu.SemaphoreType.DMA,
            pltpu.SemaphoreType.DMA,
            pltpu.SemaphoreType.DMA,
            pltpu.SemaphoreType.DMA,
            pltpu.SemaphoreType.DMA,
        ],
    )(_edge_body)
    return f(ps, pr, u, v, r, src, dst, typ, z128, z16)


def _fin_body(a128_ref, a16_ref, pd_ref, o_ref):
    s = a128_ref[0, :N] + a128_ref[1, :N]
    d = a16_ref[0, :N, :1] + a16_ref[1, :N, :1]
    safe = jnp.where(d > 0, d, jnp.float32(1.0))
    y = s / safe + pd_ref[...]
    y = jnp.where(y >= 0, y, y * jnp.float32(0.01))
    o_ref[...] = jnp.where(d > 0, y, jnp.float32(0.0))


def _finish(a128, a16, pd):
    return pl.pallas_call(
        _fin_body,
        out_shape=jax.ShapeDtypeStruct((N, D), jnp.float32),
    )(a128, a16, pd)


def kernel(x, relation_embedding, w1_w, w1_b, w2_w, edge_index, edge_type):
    ps, pd, pr, u, v, rv = _projections(x, relation_embedding, w1_w, w1_b, w2_w)
    r_pad = jnp.pad(rv[:, 0], (0, RPAD - REL))
    z128 = jnp.zeros((NPAD, D), jnp.float32)
    z16 = jnp.zeros((NPAD, 16), jnp.float32)
    a128, a16 = _edge_pass(ps, pr, u[:, 0], v[:, 0], r_pad,
                           edge_index[0], edge_index[1], edge_type, z128, z16)
    return _finish(a128, a16, pd)
